# baseline (device time: 155972 ns/iter reference)
import jax
import jax.numpy as jnp
from jax import lax
from jax.experimental import pallas as pl
from jax.experimental.pallas import tpu as pltpu

N_Z = 4
S = 1024
D = 2048
DCS = 128
H = 16
DH = 128
DR = 32
PAY = S + 2 * D
SCALE = (DH + DR) ** -0.5

_F32 = jnp.float32


def _kv_body(x_ref, wdkv_ref, wuk_ref, wuv_ref,
             k_ref, v_ref, comm_ref, send_sems, recv_sems):
    my_x = lax.axis_index("x")
    my_y = lax.axis_index("y")
    my_z = lax.axis_index("z")
    left = (my_z - 1) % N_Z
    right = (my_z + 1) % N_Z

    barrier_sem = pltpu.get_barrier_semaphore()
    for nbr in [left, right]:
        pl.semaphore_signal(
            barrier_sem, inc=1,
            device_id=(my_x, my_y, nbr),
            device_id_type=pl.DeviceIdType.MESH,
        )
    pl.semaphore_wait(barrier_sem, 2)

    xx = x_ref[0]

    cT = lax.dot_general(
        wdkv_ref[...], xx, (((0,), (1,)), ((), ())),
        preferred_element_type=_F32,
    )
    comm_ref[0, :, 0:S] = cT.astype(jnp.bfloat16)
    comm_ref[0, :, S:S + D] = wuk_ref[...].astype(jnp.bfloat16)
    comm_ref[0, :, S + D:PAY] = wuv_ref[...].astype(jnp.bfloat16)

    def acc(slot, first=False):
        chunk_cT = comm_ref[slot, :, 0:S]
        dk = lax.dot_general(
            chunk_cT, comm_ref[slot, :, S:S + D],
            (((0,), (0,)), ((), ())),
            preferred_element_type=_F32,
        )
        dv = lax.dot_general(
            chunk_cT, comm_ref[slot, :, S + D:PAY],
            (((0,), (0,)), ((), ())),
            preferred_element_type=_F32,
        )
        if first:
            k_ref[...] = dk.astype(jnp.bfloat16)
            v_ref[...] = dv.astype(jnp.bfloat16)
        else:
            k_ref[...] += dk.astype(jnp.bfloat16)
            v_ref[...] += dv.astype(jnp.bfloat16)

    for h in range(N_Z - 1):
        rdma = pltpu.make_async_remote_copy(
            src_ref=comm_ref.at[h],
            dst_ref=comm_ref.at[(h + 1) % 3],
            send_sem=send_sems.at[h],
            recv_sem=recv_sems.at[(h + 1) % 3],
            device_id=(my_x, my_y, right),
            device_id_type=pl.DeviceIdType.MESH,
        )
        rdma.start()
        acc(h, first=(h == 0))
        rdma.wait()
    acc(0)


def _proj_body(x_ref, wq_ref, wqr_ref, wkr_ref, q_ref, qr_ref, kr_ref):
    xx = x_ref[0].astype(jnp.bfloat16)
    q = jnp.dot(
        xx, wq_ref[...].astype(jnp.bfloat16), preferred_element_type=_F32
    ) * SCALE
    q_ref[...] = q.astype(jnp.bfloat16)
    qr = jnp.dot(
        xx, wqr_ref[...].astype(jnp.bfloat16), preferred_element_type=_F32
    ) * SCALE
    for h in range(H):
        qr_ref[h] = qr[:, h * DR:(h + 1) * DR].astype(jnp.bfloat16)
    kr_ref[...] = jnp.dot(
        xx, wkr_ref[...].astype(jnp.bfloat16), preferred_element_type=_F32
    ).astype(jnp.bfloat16)


def _attn_body(q_ref, k_ref, v_ref, qr_ref, kr_ref, o_ref):
    s = lax.dot_general(
        q_ref[...], k_ref[...], (((1,), (1,)), ((), ())),
        preferred_element_type=_F32,
    )
    s += lax.dot_general(
        qr_ref[0], kr_ref[...], (((1,), (1,)), ((), ())),
        preferred_element_type=_F32,
    )
    p = jnp.exp(s)
    denom = jnp.sum(p, axis=1, keepdims=True)
    o = jnp.dot(p.astype(jnp.bfloat16), v_ref[...], preferred_element_type=_F32)
    o_ref[...] = (o / denom).astype(jnp.bfloat16)


def _out_body(o_ref, wo_ref, out_ref):
    out_ref[0] = jnp.dot(
        o_ref[...], wo_ref[...].astype(jnp.bfloat16),
        preferred_element_type=_F32,
    )


def kernel(x, Wdkv, Wuk, Wuv, Wq, Wqr, Wkr, Wo):
    vmem = pl.BlockSpec(memory_space=pltpu.VMEM)

    K, V = pl.pallas_call(
        _kv_body,
        out_shape=(
            jax.ShapeDtypeStruct((S, D), jnp.bfloat16),
            jax.ShapeDtypeStruct((S, D), jnp.bfloat16),
        ),
        in_specs=[vmem] * 4,
        out_specs=(vmem, vmem),
        scratch_shapes=[
            pltpu.VMEM((3, DCS, PAY), jnp.bfloat16),
            pltpu.SemaphoreType.DMA((3,)),
            pltpu.SemaphoreType.DMA((3,)),
        ],
        compiler_params=pltpu.CompilerParams(
            collective_id=0, vmem_limit_bytes=34 * 1024 * 1024
        ),
    )(x, Wdkv, Wuk, Wuv)

    Q, Qr, Kr = pl.pallas_call(
        _proj_body,
        out_shape=(
            jax.ShapeDtypeStruct((S, D), jnp.bfloat16),
            jax.ShapeDtypeStruct((H, S, DR), jnp.bfloat16),
            jax.ShapeDtypeStruct((S, DR), jnp.bfloat16),
        ),
        in_specs=[vmem] * 4,
        out_specs=(vmem, vmem, vmem),
    )(x, Wq, Wqr, Wkr)

    O = pl.pallas_call(
        _attn_body,
        grid=(H,),
        out_shape=jax.ShapeDtypeStruct((S, D), jnp.bfloat16),
        in_specs=[
            pl.BlockSpec((S, DH), lambda h: (0, h)),
            pl.BlockSpec((S, DH), lambda h: (0, h)),
            pl.BlockSpec((S, DH), lambda h: (0, h)),
            pl.BlockSpec((1, S, DR), lambda h: (h, 0, 0)),
            pl.BlockSpec((S, DR), lambda h: (0, 0)),
        ],
        out_specs=pl.BlockSpec((S, DH), lambda h: (0, h)),
    )(Q, K, V, Qr, Kr)

    return pl.pallas_call(
        _out_body,
        out_shape=jax.ShapeDtypeStruct((1, S, D), _F32),
        in_specs=[vmem, vmem],
        out_specs=vmem,
    )(O, Wo)


# device time: 110670 ns/iter; 1.4093x vs baseline; 1.4093x over previous
import jax
import jax.numpy as jnp
from jax import lax
from jax.experimental import pallas as pl
from jax.experimental.pallas import tpu as pltpu

N_Z = 4
S = 1024
D = 2048
DCS = 128
H = 16
HB = 4
DH = 128
DR = 32
DCB = HB * DH
SCALE = (DH + DR) ** -0.5

_F32 = jnp.float32
_BF16 = jnp.bfloat16


def _kc_body(x_ref, wdkv_ref, wuk_ref, wuv_ref, kc_ref, vc_ref,
             ct_buf, wsend, ct_slots, w_slots,
             ct_ssem, ct_rsem, w_ssem, w_rsem):
    my_x = lax.axis_index("x")
    my_y = lax.axis_index("y")
    my_z = lax.axis_index("z")
    peers = [(my_z + 1) % N_Z, (my_z + 3) % N_Z, (my_z + 2) % N_Z]

    barrier_sem = pltpu.get_barrier_semaphore()
    for w in peers:
        pl.semaphore_signal(
            barrier_sem, inc=1,
            device_id=(my_x, my_y, w),
            device_id_type=pl.DeviceIdType.MESH,
        )
    pl.semaphore_wait(barrier_sem, N_Z - 1)

    ct_buf[...] = lax.dot_general(
        wdkv_ref[...].astype(_BF16), x_ref[0].astype(_BF16),
        (((0,), (1,)), ((), ())),
        preferred_element_type=_F32,
    ).astype(_BF16)

    for j, w in enumerate(peers):
        wsend[j, :, 0:DCB] = wuk_ref[:, pl.ds(w * DCB, DCB)].astype(_BF16)
        wsend[j, :, DCB:2 * DCB] = wuv_ref[:, pl.ds(w * DCB, DCB)].astype(_BF16)

    rdmas = []
    for j, w in enumerate(peers):
        for src, dst, ssem, rsem in (
            (ct_buf, ct_slots.at[my_z], ct_ssem.at[w], ct_rsem.at[my_z]),
            (wsend.at[j], w_slots.at[my_z], w_ssem.at[w], w_rsem.at[my_z]),
        ):
            rdma = pltpu.make_async_remote_copy(
                src_ref=src, dst_ref=dst, send_sem=ssem, recv_sem=rsem,
                device_id=(my_x, my_y, w),
                device_id_type=pl.DeviceIdType.MESH,
            )
            rdma.start()
            rdmas.append(rdma)

    my_wk = wuk_ref[:, pl.ds(my_z * DCB, DCB)].astype(_BF16)
    my_wv = wuv_ref[:, pl.ds(my_z * DCB, DCB)].astype(_BF16)
    ct = ct_buf[...]
    kc_ref[...] = lax.dot_general(
        ct, my_wk, (((0,), (0,)), ((), ())), preferred_element_type=_F32
    ).astype(_BF16)
    vc_ref[...] = lax.dot_general(
        ct, my_wv, (((0,), (0,)), ((), ())), preferred_element_type=_F32
    ).astype(_BF16)

    for s in peers:
        for src, dst, ssem, rsem in (
            (ct_buf, ct_slots.at[s], ct_ssem.at[s], ct_rsem.at[s]),
            (wsend.at[0], w_slots.at[s], w_ssem.at[s], w_rsem.at[s]),
        ):
            recv = pltpu.make_async_remote_copy(
                src_ref=src, dst_ref=dst, send_sem=ssem, recv_sem=rsem,
                device_id=(my_x, my_y, s),
                device_id_type=pl.DeviceIdType.MESH,
            )
            recv.wait_recv()
        kc_ref[...] += lax.dot_general(
            ct_slots[s], w_slots[s, :, 0:DCB],
            (((0,), (0,)), ((), ())), preferred_element_type=_F32,
        ).astype(_BF16)
        vc_ref[...] += lax.dot_general(
            ct_slots[s], w_slots[s, :, DCB:2 * DCB],
            (((0,), (0,)), ((), ())), preferred_element_type=_F32,
        ).astype(_BF16)

    for r in rdmas:
        r.wait_send()


def _proj_body(x_ref, wq_ref, wqr_ref, wkr_ref, q_ref, qr_ref, kr_ref):
    xx = x_ref[0].astype(_BF16)
    q = jnp.dot(
        xx, wq_ref[...].astype(_BF16), preferred_element_type=_F32
    ) * SCALE
    q_ref[...] = q.astype(_BF16)
    qr = jnp.dot(
        xx, wqr_ref[...].astype(_BF16), preferred_element_type=_F32
    ) * SCALE
    for h in range(HB):
        qr_ref[h] = qr[:, h * DR:(h + 1) * DR].astype(_BF16)
    kr_ref[...] = jnp.dot(
        xx, wkr_ref[...].astype(_BF16), preferred_element_type=_F32
    ).astype(_BF16)


def _attn_body(q_ref, k_ref, v_ref, qr_ref, kr_ref, o_ref):
    s = lax.dot_general(
        q_ref[...], k_ref[...], (((1,), (1,)), ((), ())),
        preferred_element_type=_F32,
    )
    s += lax.dot_general(
        qr_ref[0], kr_ref[...], (((1,), (1,)), ((), ())),
        preferred_element_type=_F32,
    )
    p = jnp.exp(s)
    denom = jnp.sum(p, axis=1, keepdims=True)
    o = jnp.dot(p.astype(_BF16), v_ref[...], preferred_element_type=_F32)
    o_ref[...] = (o / denom).astype(_BF16)


def _out_body(o_ref, wo_ref, out_ref, comm_ref, send_sems, recv_sems):
    my_x = lax.axis_index("x")
    my_y = lax.axis_index("y")
    my_z = lax.axis_index("z")
    left = (my_z - 1) % N_Z
    right = (my_z + 1) % N_Z

    barrier_sem = pltpu.get_barrier_semaphore()
    for nbr in [left, right]:
        pl.semaphore_signal(
            barrier_sem, inc=1,
            device_id=(my_x, my_y, nbr),
            device_id_type=pl.DeviceIdType.MESH,
        )
    pl.semaphore_wait(barrier_sem, 2)

    comm_ref[0] = o_ref[...]

    def acc(slot, origin, first=False):
        wo_rows = wo_ref[pl.ds(origin * DCB, DCB), :]
        d = jnp.dot(comm_ref[slot], wo_rows, preferred_element_type=_F32)
        if first:
            out_ref[0] = d
        else:
            out_ref[0] += d

    for h in range(N_Z - 1):
        rdma = pltpu.make_async_remote_copy(
            src_ref=comm_ref.at[h],
            dst_ref=comm_ref.at[(h + 1) % 3],
            send_sem=send_sems.at[h],
            recv_sem=recv_sems.at[(h + 1) % 3],
            device_id=(my_x, my_y, right),
            device_id_type=pl.DeviceIdType.MESH,
        )
        rdma.start()
        acc(h, (my_z - h) % N_Z, first=(h == 0))
        rdma.wait()
    acc(0, (my_z - 3) % N_Z)


def kernel(x, Wdkv, Wuk, Wuv, Wq, Wqr, Wkr, Wo):
    vmem = pl.BlockSpec(memory_space=pltpu.VMEM)
    my_z = lax.axis_index("z")

    Kc, Vc = pl.pallas_call(
        _kc_body,
        out_shape=(
            jax.ShapeDtypeStruct((S, DCB), _BF16),
            jax.ShapeDtypeStruct((S, DCB), _BF16),
        ),
        in_specs=[vmem] * 4,
        out_specs=(vmem, vmem),
        scratch_shapes=[
            pltpu.VMEM((DCS, S), _BF16),
            pltpu.VMEM((3, DCS, 2 * DCB), _BF16),
            pltpu.VMEM((N_Z, DCS, S), _BF16),
            pltpu.VMEM((N_Z, DCS, 2 * DCB), _BF16),
            pltpu.SemaphoreType.DMA((N_Z,)),
            pltpu.SemaphoreType.DMA((N_Z,)),
            pltpu.SemaphoreType.DMA((N_Z,)),
            pltpu.SemaphoreType.DMA((N_Z,)),
        ],
        compiler_params=pltpu.CompilerParams(collective_id=0),
    )(x, Wdkv, Wuk, Wuv)

    Wq_my = lax.dynamic_slice(Wq, (0, my_z * DCB), (D, DCB))
    Wqr_my = lax.dynamic_slice(Wqr, (0, my_z * HB * DR), (D, HB * DR))
    Q, Qr, Kr = pl.pallas_call(
        _proj_body,
        out_shape=(
            jax.ShapeDtypeStruct((S, DCB), _BF16),
            jax.ShapeDtypeStruct((HB, S, DR), _BF16),
            jax.ShapeDtypeStruct((S, DR), _BF16),
        ),
        in_specs=[vmem] * 4,
        out_specs=(vmem, vmem, vmem),
    )(x, Wq_my, Wqr_my, Wkr)

    O = pl.pallas_call(
        _attn_body,
        grid=(HB,),
        out_shape=jax.ShapeDtypeStruct((S, DCB), _BF16),
        in_specs=[
            pl.BlockSpec((S, DH), lambda h: (0, h)),
            pl.BlockSpec((S, DH), lambda h: (0, h)),
            pl.BlockSpec((S, DH), lambda h: (0, h)),
            pl.BlockSpec((1, S, DR), lambda h: (h, 0, 0)),
            pl.BlockSpec((S, DR), lambda h: (0, 0)),
        ],
        out_specs=pl.BlockSpec((S, DH), lambda h: (0, h)),
    )(Q, Kc, Vc, Qr, Kr)

    return pl.pallas_call(
        _out_body,
        out_shape=jax.ShapeDtypeStruct((1, S, D), _F32),
        in_specs=[vmem, vmem],
        out_specs=vmem,
        scratch_shapes=[
            pltpu.VMEM((3, S, DCB), _BF16),
            pltpu.SemaphoreType.DMA((3,)),
            pltpu.SemaphoreType.DMA((3,)),
        ],
        compiler_params=pltpu.CompilerParams(
            collective_id=1, vmem_limit_bytes=34 * 1024 * 1024
        ),
    )(O, Wo)


# device time: 98240 ns/iter; 1.5877x vs baseline; 1.1265x over previous
import jax
import jax.numpy as jnp
from jax import lax
from jax.experimental import pallas as pl
from jax.experimental.pallas import tpu as pltpu

N_Z = 4
S = 1024
D = 2048
DCS = 128
H = 16
HB = 4
DH = 128
DR = 32
DCB = HB * DH
SCALE = (DH + DR) ** -0.5

_F32 = jnp.float32
_BF16 = jnp.bfloat16


def _body(x_ref, wdkv_ref, wuk_ref, wuv_ref, wqm_ref, wqrm_ref, wkr_ref,
          o_ref,
          ct_buf, wsend, ct_slots, w_slots, kc, vc, q_buf, qr_buf, kr_buf,
          ct_ssem, ct_rsem, w_ssem, w_rsem):
    my_x = lax.axis_index("x")
    my_y = lax.axis_index("y")
    my_z = lax.axis_index("z")
    left = (my_z - 1) % N_Z
    right = (my_z + 1) % N_Z
    peers = [right, left, (my_z + 2) % N_Z]

    barrier_sem = pltpu.get_barrier_semaphore()
    for w in peers:
        pl.semaphore_signal(
            barrier_sem, inc=1,
            device_id=(my_x, my_y, w),
            device_id_type=pl.DeviceIdType.MESH,
        )
    pl.semaphore_wait(barrier_sem, N_Z - 1)

    ct_buf[...] = lax.dot_general(
        wdkv_ref[...].astype(_BF16), x_ref[0].astype(_BF16),
        (((0,), (1,)), ((), ())),
        preferred_element_type=_F32,
    ).astype(_BF16)

    for j, w in enumerate(peers):
        wsend[j, :, 0:DCB] = wuk_ref[:, pl.ds(w * DCB, DCB)].astype(_BF16)
        wsend[j, :, DCB:2 * DCB] = wuv_ref[:, pl.ds(w * DCB, DCB)].astype(_BF16)

    rdmas = []
    for j, w in enumerate(peers):
        for src, dst, ssem, rsem in (
            (ct_buf, ct_slots.at[my_z], ct_ssem.at[w], ct_rsem.at[my_z]),
            (wsend.at[j], w_slots.at[my_z], w_ssem.at[w], w_rsem.at[my_z]),
        ):
            rdma = pltpu.make_async_remote_copy(
                src_ref=src, dst_ref=dst, send_sem=ssem, recv_sem=rsem,
                device_id=(my_x, my_y, w),
                device_id_type=pl.DeviceIdType.MESH,
            )
            rdma.start()
            rdmas.append(rdma)

    xx = x_ref[0].astype(_BF16)
    q_buf[...] = (jnp.dot(
        xx, wqm_ref[...].astype(_BF16), preferred_element_type=_F32
    ) * SCALE).astype(_BF16)
    qr = jnp.dot(
        xx, wqrm_ref[...].astype(_BF16), preferred_element_type=_F32
    ) * SCALE
    for h in range(HB):
        qr_buf[h] = qr[:, h * DR:(h + 1) * DR].astype(_BF16)
    kr_buf[...] = jnp.dot(
        xx, wkr_ref[...].astype(_BF16), preferred_element_type=_F32
    ).astype(_BF16)

    my_wk = wuk_ref[:, pl.ds(my_z * DCB, DCB)].astype(_BF16)
    my_wv = wuv_ref[:, pl.ds(my_z * DCB, DCB)].astype(_BF16)
    ct = ct_buf[...]
    kc[...] = lax.dot_general(
        ct, my_wk, (((0,), (0,)), ((), ())), preferred_element_type=_F32
    ).astype(_BF16)
    vc[...] = lax.dot_general(
        ct, my_wv, (((0,), (0,)), ((), ())), preferred_element_type=_F32
    ).astype(_BF16)

    for s in peers:
        for src, dst, ssem, rsem in (
            (ct_buf, ct_slots.at[s], ct_ssem.at[s], ct_rsem.at[s]),
            (wsend.at[0], w_slots.at[s], w_ssem.at[s], w_rsem.at[s]),
        ):
            recv = pltpu.make_async_remote_copy(
                src_ref=src, dst_ref=dst, send_sem=ssem, recv_sem=rsem,
                device_id=(my_x, my_y, s),
                device_id_type=pl.DeviceIdType.MESH,
            )
            recv.wait_recv()
        kc[...] += lax.dot_general(
            ct_slots[s], w_slots[s, :, 0:DCB],
            (((0,), (0,)), ((), ())), preferred_element_type=_F32,
        ).astype(_BF16)
        vc[...] += lax.dot_general(
            ct_slots[s], w_slots[s, :, DCB:2 * DCB],
            (((0,), (0,)), ((), ())), preferred_element_type=_F32,
        ).astype(_BF16)

    for h in range(HB):
        cols = slice(h * DH, (h + 1) * DH)
        s_scores = lax.dot_general(
            q_buf[:, cols], kc[:, cols], (((1,), (1,)), ((), ())),
            preferred_element_type=_F32,
        )
        s_scores += lax.dot_general(
            qr_buf[h], kr_buf[...], (((1,), (1,)), ((), ())),
            preferred_element_type=_F32,
        )
        p = jnp.exp(s_scores)
        denom = jnp.sum(p, axis=1, keepdims=True)
        o = jnp.dot(p.astype(_BF16), vc[:, cols], preferred_element_type=_F32)
        o_ref[:, cols] = (o / denom).astype(_BF16)

    for r in rdmas:
        r.wait_send()


def _out_body(o_ref, wo_ref, out_ref, comm_ref, send_sems, recv_sems):
    my_x = lax.axis_index("x")
    my_y = lax.axis_index("y")
    my_z = lax.axis_index("z")
    left = (my_z - 1) % N_Z
    right = (my_z + 1) % N_Z

    barrier_sem = pltpu.get_barrier_semaphore()
    for nbr in [left, right]:
        pl.semaphore_signal(
            barrier_sem, inc=1,
            device_id=(my_x, my_y, nbr),
            device_id_type=pl.DeviceIdType.MESH,
        )
    pl.semaphore_wait(barrier_sem, 2)

    comm_ref[0] = o_ref[...]

    def acc(slot, origin, first=False):
        wo_rows = wo_ref[pl.ds(origin * DCB, DCB), :]
        d = jnp.dot(comm_ref[slot], wo_rows, preferred_element_type=_F32)
        if first:
            out_ref[0] = d
        else:
            out_ref[0] += d

    for h in range(N_Z - 1):
        rdma = pltpu.make_async_remote_copy(
            src_ref=comm_ref.at[h],
            dst_ref=comm_ref.at[(h + 1) % 3],
            send_sem=send_sems.at[h],
            recv_sem=recv_sems.at[(h + 1) % 3],
            device_id=(my_x, my_y, right),
            device_id_type=pl.DeviceIdType.MESH,
        )
        rdma.start()
        acc(h, (my_z - h) % N_Z, first=(h == 0))
        rdma.wait()
    acc(0, (my_z - 3) % N_Z)


def kernel(x, Wdkv, Wuk, Wuv, Wq, Wqr, Wkr, Wo):
    vmem = pl.BlockSpec(memory_space=pltpu.VMEM)
    my_z = lax.axis_index("z")

    Wq_my = lax.dynamic_slice(Wq, (0, my_z * DCB), (D, DCB))
    Wqr_my = lax.dynamic_slice(Wqr, (0, my_z * HB * DR), (D, HB * DR))

    O = pl.pallas_call(
        _body,
        out_shape=jax.ShapeDtypeStruct((S, DCB), _BF16),
        in_specs=[vmem] * 7,
        out_specs=vmem,
        scratch_shapes=[
            pltpu.VMEM((DCS, S), _BF16),
            pltpu.VMEM((3, DCS, 2 * DCB), _BF16),
            pltpu.VMEM((N_Z, DCS, S), _BF16),
            pltpu.VMEM((N_Z, DCS, 2 * DCB), _BF16),
            pltpu.VMEM((S, DCB), _BF16),
            pltpu.VMEM((S, DCB), _BF16),
            pltpu.VMEM((S, DCB), _BF16),
            pltpu.VMEM((HB, S, DR), _BF16),
            pltpu.VMEM((S, DR), _BF16),
            pltpu.SemaphoreType.DMA((N_Z,)),
            pltpu.SemaphoreType.DMA((N_Z,)),
            pltpu.SemaphoreType.DMA((N_Z,)),
            pltpu.SemaphoreType.DMA((N_Z,)),
        ],
        compiler_params=pltpu.CompilerParams(
            collective_id=0, vmem_limit_bytes=36 * 1024 * 1024
        ),
    )(x, Wdkv, Wuk, Wuv, Wq_my, Wqr_my, Wkr)

    return pl.pallas_call(
        _out_body,
        out_shape=jax.ShapeDtypeStruct((1, S, D), _F32),
        in_specs=[vmem, vmem],
        out_specs=vmem,
        scratch_shapes=[
            pltpu.VMEM((3, S, DCB), _BF16),
            pltpu.SemaphoreType.DMA((3,)),
            pltpu.SemaphoreType.DMA((3,)),
        ],
        compiler_params=pltpu.CompilerParams(
            collective_id=1, vmem_limit_bytes=34 * 1024 * 1024
        ),
    )(O, Wo)
